# baseline (speedup 1.0000x reference)
import functools

import jax
import jax.numpy as jnp
from jax import lax
from jax.experimental import pallas as pl
from jax.experimental.pallas import tpu as pltpu
from jax.experimental.pallas import tpu_sc as plsc

_NC = 2
_NS = 16


def _body(N, k, D, n_b, idx_hbm, x_hbm, out_hbm,
          idx_b, rowbuf, table, sem0, sem1, gsem):
    cid = lax.axis_index("c")
    sid = lax.axis_index("s")
    rows_per_tile = N // _NS
    k_per_tile = k // _NS
    stage_sems = (sem0, sem1)

    def stage(bb):
        b = cid * n_b + bb
        return pltpu.async_copy(
            x_hbm.at[b, pl.ds(sid * rows_per_tile, rows_per_tile)],
            table.at[pl.ds(sid * rows_per_tile, rows_per_tile)],
            stage_sems[bb % 2],
        )

    pending = stage(0)
    for bb in range(n_b):
        b = cid * n_b + bb
        pltpu.sync_copy(idx_hbm.at[cid, sid, bb], idx_b)
        pending.wait()
        plsc.subcore_barrier()
        pltpu.async_copy(table.at[idx_b], rowbuf, gsem).wait()
        plsc.subcore_barrier()
        if bb + 1 < n_b:
            pending = stage(bb + 1)
        pltpu.sync_copy(rowbuf.at[pl.ds(0, k_per_tile)],
                        out_hbm.at[b, pl.ds(sid * k_per_tile, k_per_tile)])


def kernel(topk_idx, x):
    B, N, D = x.shape
    k = topk_idx.shape[1]
    n_b = B // _NC

    idx = jnp.repeat((topk_idx.astype(jnp.int32) * 2)
                     .reshape(_NC, n_b, _NS, k // _NS)
                     .transpose(0, 2, 1, 3), 2, axis=-1)

    mesh = plsc.VectorSubcoreMesh(core_axis_name="c", subcore_axis_name="s")
    run = pl.kernel(
        functools.partial(_body, N, k, D, n_b),
        mesh=mesh,
        out_type=jax.ShapeDtypeStruct((B, k, D), x.dtype),
        scratch_types=[
            pltpu.VMEM((2 * (k // _NS),), jnp.int32),
            pltpu.VMEM((2 * (k // _NS), D), jnp.float32),
            pltpu.VMEM_SHARED((2 * N - 1, D), jnp.float32),
            pltpu.SemaphoreType.DMA,
            pltpu.SemaphoreType.DMA,
            pltpu.SemaphoreType.DMA,
        ],
    )
    return run(idx, x)


# speedup vs baseline: 1.7048x; 1.7048x over previous
import functools

import jax
import jax.numpy as jnp
from jax import lax
from jax.experimental import pallas as pl
from jax.experimental.pallas import tpu as pltpu
from jax.experimental.pallas import tpu_sc as plsc

_NC = 2
_NS = 16
_NW = _NC * _NS
_SUB = 1024
_CH = 256
_NSEM = 4


def _body(k, idx_hbm, x_hbm, out_hbm, idx_v, rowbuf, s0, s1, s2, s3):
    sems = (s0, s1, s2, s3)
    wid = lax.axis_index("s") * _NC + lax.axis_index("c")
    n_sub_per_b = k // _SUB
    for half in range(2):
        b = wid * 2 + half
        for sub in range(n_sub_per_b):
            pltpu.sync_copy(idx_hbm.at[wid, half * n_sub_per_b + sub], idx_v)
            for ch in range(_SUB // _CH):
                def issue(g, carry):
                    vec = idx_v[pl.ds(ch * _CH + g * 16, 16)]
                    for l in range(16):
                        pltpu.async_copy(
                            x_hbm.at[b, vec[l]], rowbuf.at[g * 16 + l],
                            sems[l % _NSEM])
                    return carry
                lax.fori_loop(0, _CH // 16, issue, 0)
                for q in range(_NSEM):
                    pltpu.make_async_copy(
                        x_hbm.at[b, pl.ds(0, _CH // _NSEM)],
                        rowbuf.at[pl.ds(q * (_CH // _NSEM), _CH // _NSEM)],
                        sems[q]).wait()
                pltpu.sync_copy(
                    rowbuf,
                    out_hbm.at[b, pl.ds((sub * (_SUB // _CH) + ch) * _CH, _CH)])


def kernel(topk_idx, x):
    B, N, D = x.shape
    k = topk_idx.shape[1]

    idx = topk_idx.astype(jnp.int32).reshape(_NW, (B * k) // (_NW * _SUB), _SUB)

    mesh = plsc.VectorSubcoreMesh(core_axis_name="c", subcore_axis_name="s")
    run = pl.kernel(
        functools.partial(_body, k),
        mesh=mesh,
        out_type=jax.ShapeDtypeStruct((B, k, D), x.dtype),
        scratch_types=[
            pltpu.VMEM((_SUB,), jnp.int32),
            pltpu.VMEM((_CH, D), jnp.float32),
            pltpu.SemaphoreType.DMA,
            pltpu.SemaphoreType.DMA,
            pltpu.SemaphoreType.DMA,
            pltpu.SemaphoreType.DMA,
        ],
    )
    return run(idx, x)
